# baseline (speedup 1.0000x reference)
import functools

import jax
import jax.numpy as jnp
from jax import lax
from jax.experimental import pallas as pl
from jax.experimental.pallas import tpu as pltpu
from jax.experimental.pallas import tpu_sc as plsc

_EPS = 1e-5
_K_CHUNKS = 8
_RB = 16


def _sc_gather_rows(W_word, ids_flat):
    (N,) = ids_flat.shape
    D = W_word.shape[1]
    mesh = plsc.VectorSubcoreMesh(core_axis_name="c", subcore_axis_name="s")
    NC, NS = mesh.num_cores, mesh.num_subcores
    NW = NC * NS
    n_per_w = N // NW
    chunk = 256
    n_chunks = n_per_w // chunk

    @functools.partial(
        pl.kernel,
        mesh=mesh,
        out_type=jax.ShapeDtypeStruct((N, D), jnp.float32),
        scratch_types=[
            pltpu.VMEM((chunk,), jnp.int32),
            pltpu.VMEM((chunk,), jnp.int32),
            pltpu.VMEM((chunk, D), jnp.float32),
            pltpu.VMEM((chunk, D), jnp.float32),
            pltpu.SemaphoreType.DMA,
            pltpu.SemaphoreType.DMA,
            pltpu.SemaphoreType.DMA,
            pltpu.SemaphoreType.DMA,
            pltpu.SemaphoreType.DMA,
            pltpu.SemaphoreType.DMA,
        ],
    )
    def gather_kernel(table_hbm, idx_hbm, out_hbm,
                      idx0, idx1, rows0, rows1,
                      isem0, isem1, gsem0, gsem1, osem0, osem1):
        idx_v = (idx0, idx1)
        rows_v = (rows0, rows1)
        isem = (isem0, isem1)
        gsem = (gsem0, gsem1)
        osem = (osem0, osem1)
        wid = lax.axis_index("s") * NC + lax.axis_index("c")
        base = wid * n_per_w

        def idx_slice(i):
            return idx_hbm.at[pl.ds(base + i * chunk, chunk)]

        def out_slice(i):
            return out_hbm.at[pl.ds(base + i * chunk, chunk)]

        pltpu.async_copy(idx_slice(0), idx_v[0], isem[0])
        pltpu.async_copy(idx_slice(1), idx_v[1], isem[1])

        @pl.loop(0, n_chunks, step=2)
        def _(i0):
            for b in range(2):
                i = i0 + b
                o = 1 - b
                pltpu.make_async_copy(idx_slice(i), idx_v[b], isem[b]).wait()

                @pl.when(i >= 2)
                def _():
                    pltpu.make_async_copy(
                        rows_v[b], out_slice(i - 2), osem[b]).wait()

                pltpu.async_copy(table_hbm.at[idx_v[b]], rows_v[b], gsem[b])

                @pl.when(i >= 1)
                def _():
                    pltpu.make_async_copy(
                        table_hbm.at[idx_v[o]], rows_v[o], gsem[o]).wait()
                    pltpu.async_copy(rows_v[o], out_slice(i - 1), osem[o])

                    @pl.when(i + 1 < n_chunks)
                    def _():
                        pltpu.async_copy(idx_slice(i + 1), idx_v[o], isem[o])

        last = n_chunks - 1
        bl = last % 2
        pltpu.make_async_copy(
            table_hbm.at[idx_v[bl]], rows_v[bl], gsem[bl]).wait()
        pltpu.async_copy(rows_v[bl], out_slice(last), osem[bl])
        pltpu.make_async_copy(
            rows_v[1 - bl], out_slice(last - 1), osem[1 - bl]).wait()
        pltpu.make_async_copy(rows_v[bl], out_slice(last), osem[bl]).wait()

    return gather_kernel(W_word, ids_flat)


def _finalize_body(g_ref, s_ref, w_ref, p_ref, gam_ref, bet_ref, *rest,
                   scale):
    o_ref = rest[-1]
    g = g_ref[...]
    sid = s_ref[...][..., None]
    w0 = w_ref[0:1, :][None]
    w1 = w_ref[1:2, :][None]
    seg = jnp.where(sid == 0, w0, w1)
    emb = scale * (g + seg) + p_ref[...]
    mu = jnp.mean(emb, axis=-1, keepdims=True)
    m2 = jnp.mean(emb * emb, axis=-1, keepdims=True)
    var = m2 - mu * mu
    normed = (emb - mu) * lax.rsqrt(var + _EPS)
    o_ref[...] = normed * gam_ref[...][None] + bet_ref[...][None]


def _tc_finalize_chunk(gathered, seg, W_seg, pos3, g2, b2, prev, k0, B):
    Bc, L, D = gathered.shape
    S = Bc // _RB
    base = k0 * S
    in_specs = [
        pl.BlockSpec((_RB, L, D), lambda i: (i, 0, 0)),
        pl.BlockSpec((_RB, L), lambda i: (i, 0)),
        pl.BlockSpec((2, D), lambda i: (0, 0)),
        pl.BlockSpec((1, L, D), lambda i: (0, 0, 0)),
        pl.BlockSpec((1, D), lambda i: (0, 0)),
        pl.BlockSpec((1, D), lambda i: (0, 0)),
    ]
    args = [gathered, seg, W_seg, pos3, g2, b2]
    io_alias = {}
    if prev is not None:
        in_specs.append(pl.BlockSpec((_RB, L, D), lambda i: (0, 0, 0)))
        args.append(prev)
        io_alias = {6: 0}
    return pl.pallas_call(
        functools.partial(_finalize_body, scale=float(D ** 0.5)),
        grid=(S,),
        in_specs=in_specs,
        out_specs=pl.BlockSpec((_RB, L, D), lambda i: (base + i, 0, 0)),
        out_shape=jax.ShapeDtypeStruct((B, L, D), jnp.float32),
        input_output_aliases=io_alias,
    )(*args)


def kernel(input_ids, segment_ids, W_word, W_seg, pos_table, gamma, beta):
    B, L = input_ids.shape
    D = W_word.shape[1]
    K = _K_CHUNKS
    Bc = B // K
    ids32 = input_ids.astype(jnp.int32)
    seg32 = segment_ids.astype(jnp.int32)
    pos3 = pos_table.reshape(1, L, D)
    g2 = gamma.reshape(1, D)
    b2 = beta.reshape(1, D)
    out = None
    for k in range(K):
        ids_k = ids32[k * Bc:(k + 1) * Bc].reshape(-1)
        rows = _sc_gather_rows(W_word, ids_k)
        out = _tc_finalize_chunk(
            rows.reshape(Bc, L, D), seg32[k * Bc:(k + 1) * Bc],
            W_seg, pos3, g2, b2, prev=out, k0=k, B=B)
    return out


# speedup vs baseline: 1.0316x; 1.0316x over previous
import functools

import jax
import jax.numpy as jnp
from jax import lax
from jax.experimental import pallas as pl
from jax.experimental.pallas import tpu as pltpu
from jax.experimental.pallas import tpu_sc as plsc

_EPS = 1e-5
_K_CHUNKS = 4
_RB = 16


def _sc_gather_rows(W_word, ids_flat):
    (N,) = ids_flat.shape
    D = W_word.shape[1]
    mesh = plsc.VectorSubcoreMesh(core_axis_name="c", subcore_axis_name="s")
    NC, NS = mesh.num_cores, mesh.num_subcores
    NW = NC * NS
    n_per_w = N // NW
    chunk = 256
    n_chunks = n_per_w // chunk

    @functools.partial(
        pl.kernel,
        mesh=mesh,
        out_type=jax.ShapeDtypeStruct((N, D), jnp.float32),
        scratch_types=[
            pltpu.VMEM((chunk,), jnp.int32),
            pltpu.VMEM((chunk,), jnp.int32),
            pltpu.VMEM((chunk, D), jnp.float32),
            pltpu.VMEM((chunk, D), jnp.float32),
            pltpu.SemaphoreType.DMA,
            pltpu.SemaphoreType.DMA,
            pltpu.SemaphoreType.DMA,
            pltpu.SemaphoreType.DMA,
            pltpu.SemaphoreType.DMA,
            pltpu.SemaphoreType.DMA,
        ],
    )
    def gather_kernel(table_hbm, idx_hbm, out_hbm,
                      idx0, idx1, rows0, rows1,
                      isem0, isem1, gsem0, gsem1, osem0, osem1):
        idx_v = (idx0, idx1)
        rows_v = (rows0, rows1)
        isem = (isem0, isem1)
        gsem = (gsem0, gsem1)
        osem = (osem0, osem1)
        wid = lax.axis_index("s") * NC + lax.axis_index("c")
        base = wid * n_per_w

        def idx_slice(i):
            return idx_hbm.at[pl.ds(base + i * chunk, chunk)]

        def out_slice(i):
            return out_hbm.at[pl.ds(base + i * chunk, chunk)]

        pltpu.async_copy(idx_slice(0), idx_v[0], isem[0])
        pltpu.async_copy(idx_slice(1), idx_v[1], isem[1])

        @pl.loop(0, n_chunks, step=2)
        def _(i0):
            for b in range(2):
                i = i0 + b
                o = 1 - b
                pltpu.make_async_copy(idx_slice(i), idx_v[b], isem[b]).wait()

                @pl.when(i >= 2)
                def _():
                    pltpu.make_async_copy(
                        rows_v[b], out_slice(i - 2), osem[b]).wait()

                pltpu.async_copy(table_hbm.at[idx_v[b]], rows_v[b], gsem[b])

                @pl.when(i >= 1)
                def _():
                    pltpu.make_async_copy(
                        table_hbm.at[idx_v[o]], rows_v[o], gsem[o]).wait()
                    pltpu.async_copy(rows_v[o], out_slice(i - 1), osem[o])

                    @pl.when(i + 1 < n_chunks)
                    def _():
                        pltpu.async_copy(idx_slice(i + 1), idx_v[o], isem[o])

        last = n_chunks - 1
        bl = last % 2
        pltpu.make_async_copy(
            table_hbm.at[idx_v[bl]], rows_v[bl], gsem[bl]).wait()
        pltpu.async_copy(rows_v[bl], out_slice(last), osem[bl])
        pltpu.make_async_copy(
            rows_v[1 - bl], out_slice(last - 1), osem[1 - bl]).wait()
        pltpu.make_async_copy(rows_v[bl], out_slice(last), osem[bl]).wait()

    return gather_kernel(W_word, ids_flat)


def _finalize_body(g_ref, s_ref, w_ref, p_ref, gam_ref, bet_ref, *rest,
                   scale):
    o_ref = rest[-1]
    g = g_ref[...]
    sid = s_ref[...][..., None]
    w0 = w_ref[0:1, :][None]
    w1 = w_ref[1:2, :][None]
    seg = jnp.where(sid == 0, w0, w1)
    emb = scale * (g + seg) + p_ref[...]
    mu = jnp.mean(emb, axis=-1, keepdims=True)
    m2 = jnp.mean(emb * emb, axis=-1, keepdims=True)
    var = m2 - mu * mu
    normed = (emb - mu) * lax.rsqrt(var + _EPS)
    o_ref[...] = normed * gam_ref[...][None] + bet_ref[...][None]


def _tc_finalize_chunk(gathered, seg, W_seg, pos3, g2, b2, prev, k0, B):
    Bc, L, D = gathered.shape
    S = Bc // _RB
    base = k0 * S
    in_specs = [
        pl.BlockSpec((_RB, L, D), lambda i: (i, 0, 0)),
        pl.BlockSpec((_RB, L), lambda i: (i, 0)),
        pl.BlockSpec((2, D), lambda i: (0, 0)),
        pl.BlockSpec((1, L, D), lambda i: (0, 0, 0)),
        pl.BlockSpec((1, D), lambda i: (0, 0)),
        pl.BlockSpec((1, D), lambda i: (0, 0)),
    ]
    args = [gathered, seg, W_seg, pos3, g2, b2]
    io_alias = {}
    if prev is not None:
        in_specs.append(pl.BlockSpec((_RB, L, D), lambda i: (0, 0, 0)))
        args.append(prev)
        io_alias = {6: 0}
    return pl.pallas_call(
        functools.partial(_finalize_body, scale=float(D ** 0.5)),
        grid=(S,),
        in_specs=in_specs,
        out_specs=pl.BlockSpec((_RB, L, D), lambda i: (base + i, 0, 0)),
        out_shape=jax.ShapeDtypeStruct((B, L, D), jnp.float32),
        input_output_aliases=io_alias,
    )(*args)


def kernel(input_ids, segment_ids, W_word, W_seg, pos_table, gamma, beta):
    B, L = input_ids.shape
    D = W_word.shape[1]
    K = _K_CHUNKS
    Bc = B // K
    ids32 = input_ids.astype(jnp.int32)
    seg32 = segment_ids.astype(jnp.int32)
    pos3 = pos_table.reshape(1, L, D)
    g2 = gamma.reshape(1, D)
    b2 = beta.reshape(1, D)
    out = None
    for k in range(K):
        ids_k = ids32[k * Bc:(k + 1) * Bc].reshape(-1)
        rows = _sc_gather_rows(W_word, ids_k)
        out = _tc_finalize_chunk(
            rows.reshape(Bc, L, D), seg32[k * Bc:(k + 1) * Bc],
            W_seg, pos3, g2, b2, prev=out, k0=k, B=B)
    return out
